# baseline (speedup 1.0000x reference)
import functools

import jax
import jax.numpy as jnp
from jax.experimental import pallas as pl
from jax.experimental.pallas import tpu as pltpu


def _kmeans_loss_body(x_ref, mu_ref, o_ref, musq_ref, *, inv_g):
    mb = mu_ref[...]

    @pl.when(pl.program_id(0) == 0)
    def _():
        m32 = mb.astype(jnp.float32)
        musq_ref[...] = jnp.sum(m32 * m32, axis=0, keepdims=True)

    x = x_ref[...]
    xb = x.astype(jnp.bfloat16)
    dot = jnp.dot(xb, mb, preferred_element_type=jnp.float32)
    xsq = jnp.sum(x * x, axis=1)
    d = musq_ref[...] - 2.0 * dot
    mins = jnp.min(d, axis=1) + xsq
    o_ref[...] = (mins * inv_g)[:, None]


def kernel(images, mu):
    B, G = images.shape
    _, K = mu.shape
    mu_bf = mu.astype(jnp.bfloat16)
    nb = 8
    bb = B // nb
    out = pl.pallas_call(
        functools.partial(_kmeans_loss_body, inv_g=1.0 / G),
        out_shape=jax.ShapeDtypeStruct((B, 1), jnp.float32),
        grid=(nb,),
        in_specs=[
            pl.BlockSpec((bb, G), lambda i: (i, 0)),
            pl.BlockSpec((G, K), lambda i: (0, 0)),
        ],
        out_specs=pl.BlockSpec((bb, 1), lambda i: (i, 0)),
        scratch_shapes=[pltpu.VMEM((1, K), jnp.float32)],
        compiler_params=pltpu.CompilerParams(
            dimension_semantics=("arbitrary",),
        ),
    )(images, mu_bf)
    return out[:, 0]


# speedup vs baseline: 2.5155x; 2.5155x over previous
import functools

import jax
import jax.numpy as jnp
from jax.experimental import pallas as pl


def _kmeans_loss_body(x_ref, mu_ref, o_ref, *, inv_g):
    x = x_ref[...]
    m = mu_ref[...]
    dot = jnp.dot(x.astype(jnp.bfloat16), m.astype(jnp.bfloat16),
                  preferred_element_type=jnp.float32)
    musq = jnp.sum(m * m, axis=0)
    xsq = jnp.sum(x * x, axis=1)
    d = musq[None, :] - 2.0 * dot
    mins = jnp.min(d, axis=1) + xsq
    o_ref[...] = (mins * inv_g)[None, :]


def kernel(images, mu):
    B, G = images.shape
    _, K = mu.shape
    out = pl.pallas_call(
        functools.partial(_kmeans_loss_body, inv_g=1.0 / G),
        out_shape=jax.ShapeDtypeStruct((1, B), jnp.float32),
        grid=(1,),
        in_specs=[
            pl.BlockSpec((B, G), lambda i: (0, 0)),
            pl.BlockSpec((G, K), lambda i: (0, 0)),
        ],
        out_specs=pl.BlockSpec((1, B), lambda i: (0, 0)),
    )(images, mu)
    return out.reshape(B)


# speedup vs baseline: 2.9248x; 1.1627x over previous
import functools

import jax
import jax.numpy as jnp
from jax.experimental import pallas as pl


def _kmeans_loss_body(x_ref, mu_ref, o_ref, *, inv_g):
    x = x_ref[...]
    m = mu_ref[...]
    b = x.shape[0]
    bf = jnp.bfloat16
    musq = jnp.sum(m * m, axis=0, keepdims=True)
    ones_row = jnp.ones((1, m.shape[1]), bf)
    lhs = jnp.concatenate(
        [m.astype(bf), (-0.5 * musq).astype(bf), ones_row], axis=0)
    xsq = jnp.sum(x * x, axis=1, keepdims=True)
    ones_col = jnp.ones((b, 1), bf)
    rhs = jnp.concatenate(
        [x.astype(bf), ones_col, (-0.5 * xsq).astype(bf)], axis=1)
    p = jax.lax.dot_general(
        lhs, rhs, (((0,), (1,)), ((), ())),
        preferred_element_type=jnp.float32)
    pmax = jnp.max(p, axis=0, keepdims=True)
    o_ref[...] = pmax * (-2.0 * inv_g)


def kernel(images, mu):
    B, G = images.shape
    _, K = mu.shape
    out = pl.pallas_call(
        functools.partial(_kmeans_loss_body, inv_g=1.0 / G),
        out_shape=jax.ShapeDtypeStruct((1, B), jnp.float32),
        grid=(1,),
        in_specs=[
            pl.BlockSpec((B, G), lambda i: (0, 0)),
            pl.BlockSpec((G, K), lambda i: (0, 0)),
        ],
        out_specs=pl.BlockSpec((1, B), lambda i: (0, 0)),
    )(images, mu)
    return out.reshape(B)
